# baseline (speedup 1.0000x reference)
import functools

import jax
import jax.numpy as jnp
from jax import lax
from jax.experimental import pallas as pl
from jax.experimental.pallas import tpu as pltpu
from jax.experimental.pallas import tpu_sc as plsc

_B = 16384
_L = 50
_D = 32
_NC = 2
_NS = 16
_NW = _NC * _NS
_BPW = _B // _NW
_CB = 32
_NCHUNK = _BPW // _CB
_CL = _CB * _L
_SLICES = [(j * 128, min(128, _CL - j * 128)) for j in range((_CL + 127) // 128)]

_mesh = plsc.VectorSubcoreMesh(core_axis_name="c", subcore_axis_name="s")


@functools.partial(
    pl.kernel,
    out_type=jax.ShapeDtypeStruct((_B, 4 * _D), jnp.float32),
    mesh=_mesh,
    scratch_types=[
        pltpu.VMEM((2, _CL), jnp.int32),
        pltpu.VMEM((2, _CL, _D), jnp.float32),
        pltpu.VMEM((2, _CB), jnp.int32),
        pltpu.VMEM((2, _CB), jnp.int32),
        pltpu.VMEM((2, _CB), jnp.int32),
        pltpu.VMEM((2, _CB, _D), jnp.float32),
        pltpu.VMEM((2, _CB, _D), jnp.float32),
        pltpu.VMEM((2, _CB, _D), jnp.float32),
        pltpu.VMEM((_CB, 4 * _D), jnp.float32),
        pltpu.SemaphoreType.DMA,
        pltpu.SemaphoreType.DMA,
    ],
    compiler_params=pltpu.CompilerParams(use_tc_tiling_on_sc=False),
)
def _sc_embed(uid_hbm, age_hbm, gen_hbm, clicks_hbm, ut_hbm, at_hbm, gt_hbm,
              it_hbm, out_hbm, idx_v, rows_v, uidx_v, aidx_v, gidx_v,
              urows_v, arows_v, grows_v, outc_v, sem0, sem1):
    wid = lax.axis_index("s") * _NC + lax.axis_index("c")
    base = wid * _BPW
    inv = jnp.full((16,), 1.0 / _L, dtype=jnp.float32)
    sems = (sem0, sem1)

    def stage_and_fire(n, p, sem):
        rowbase = pl.multiple_of(base + n * _CB, _CB)
        ibase = pl.multiple_of(rowbase * _L, _CL)
        pltpu.sync_copy(clicks_hbm.at[pl.ds(ibase, _CL)], idx_v.at[p])
        for off, ln in _SLICES:
            pltpu.async_copy(
                it_hbm.at[idx_v.at[p].at[pl.ds(off, ln)]],
                rows_v.at[p].at[pl.ds(off, ln)], sem)
        pltpu.sync_copy(uid_hbm.at[pl.ds(rowbase, _CB)], uidx_v.at[p])
        pltpu.sync_copy(age_hbm.at[pl.ds(rowbase, _CB)], aidx_v.at[p])
        pltpu.sync_copy(gen_hbm.at[pl.ds(rowbase, _CB)], gidx_v.at[p])
        pltpu.async_copy(ut_hbm.at[uidx_v.at[p]], urows_v.at[p], sem)
        pltpu.async_copy(at_hbm.at[aidx_v.at[p]], arows_v.at[p], sem)
        pltpu.async_copy(gt_hbm.at[gidx_v.at[p]], grows_v.at[p], sem)

    def drain(p, sem):
        for off, ln in _SLICES:
            pltpu.make_async_copy(
                it_hbm.at[pl.ds(0, ln)], rows_v.at[p].at[pl.ds(off, ln)],
                sem).wait()
        pltpu.make_async_copy(ut_hbm.at[pl.ds(0, _CB)], urows_v.at[p],
                              sem).wait()
        pltpu.make_async_copy(ut_hbm.at[pl.ds(0, _CB)], arows_v.at[p],
                              sem).wait()
        pltpu.make_async_copy(ut_hbm.at[pl.ds(0, _CB)], grows_v.at[p],
                              sem).wait()

    def pool_and_store(n, p):
        rowbase = pl.multiple_of(base + n * _CB, _CB)

        def row_body(r, rcarry):
            rb = r * _L
            a0 = jnp.zeros((16,), jnp.float32)
            b0 = jnp.zeros((16,), jnp.float32)
            a1 = jnp.zeros((16,), jnp.float32)
            b1 = jnp.zeros((16,), jnp.float32)
            for jj in range(_L):
                lo = rows_v[p, rb + jj, pl.ds(0, 16)]
                hi = rows_v[p, rb + jj, pl.ds(16, 16)]
                if jj % 2 == 0:
                    a0 = a0 + lo
                    a1 = a1 + hi
                else:
                    b0 = b0 + lo
                    b1 = b1 + hi
            outc_v[r, pl.ds(0, 16)] = urows_v[p, r, pl.ds(0, 16)]
            outc_v[r, pl.ds(16, 16)] = urows_v[p, r, pl.ds(16, 16)]
            outc_v[r, pl.ds(32, 16)] = arows_v[p, r, pl.ds(0, 16)]
            outc_v[r, pl.ds(48, 16)] = arows_v[p, r, pl.ds(16, 16)]
            outc_v[r, pl.ds(64, 16)] = grows_v[p, r, pl.ds(0, 16)]
            outc_v[r, pl.ds(80, 16)] = grows_v[p, r, pl.ds(16, 16)]
            outc_v[r, pl.ds(96, 16)] = (a0 + b0) * inv
            outc_v[r, pl.ds(112, 16)] = (a1 + b1) * inv
            return rcarry

        lax.fori_loop(0, _CB, row_body, 0)
        pltpu.sync_copy(outc_v, out_hbm.at[pl.ds(rowbase, _CB)])

    stage_and_fire(0, 0, sems[0])

    def pair_body(t, carry):
        n0 = t * 2
        stage_and_fire(n0 + 1, 1, sems[1])
        drain(0, sems[0])
        pool_and_store(n0, 0)

        @pl.when(t < _NCHUNK // 2 - 1)
        def _():
            stage_and_fire(n0 + 2, 0, sems[0])

        drain(1, sems[1])
        pool_and_store(n0 + 1, 1)
        return carry

    lax.fori_loop(0, _NCHUNK // 2, pair_body, 0)


def _dense_body(x_ref, w_ref, b_ref, o_ref):
    acc = jnp.dot(x_ref[...], w_ref[...], preferred_element_type=jnp.float32)
    o_ref[...] = jnp.maximum(acc + b_ref[...], 0.0)


_BM = 1024
_dense = pl.pallas_call(
    _dense_body,
    grid=(_B // _BM,),
    in_specs=[
        pl.BlockSpec((_BM, 4 * _D), lambda i: (i, 0)),
        pl.BlockSpec((4 * _D, 64), lambda i: (0, 0)),
        pl.BlockSpec((1, 64), lambda i: (0, 0)),
    ],
    out_specs=pl.BlockSpec((_BM, 64), lambda i: (i, 0)),
    out_shape=jax.ShapeDtypeStruct((_B, 64), jnp.float32),
)


def kernel(user_id, age, gender, recent_clicks, user_table, age_table,
           gender_table, item_table, W, b):
    clicks_flat = recent_clicks.reshape(_B * _L)
    concat = _sc_embed(user_id, age, gender, clicks_flat, user_table,
                       age_table, gender_table, item_table)
    return _dense(concat, W, b.reshape(1, 64))


# speedup vs baseline: 1.0046x; 1.0046x over previous
import functools

import jax
import jax.numpy as jnp
from jax import lax
from jax.experimental import pallas as pl
from jax.experimental.pallas import tpu as pltpu
from jax.experimental.pallas import tpu_sc as plsc

_B = 16384
_L = 50
_D = 32
_NC = 2
_NS = 16
_NW = _NC * _NS
_BPW = _B // _NW
_CB = 32
_NCHUNK = _BPW // _CB
_CL = _CB * _L
_SLICES = [(j * 128, min(128, _CL - j * 128)) for j in range((_CL + 127) // 128)]

_mesh = plsc.VectorSubcoreMesh(core_axis_name="c", subcore_axis_name="s")


@functools.partial(
    pl.kernel,
    out_type=jax.ShapeDtypeStruct((_B, 4 * _D), jnp.float32),
    mesh=_mesh,
    scratch_types=[
        pltpu.VMEM((2, _CL), jnp.int32),
        pltpu.VMEM((2, _CL, _D), jnp.float32),
        pltpu.VMEM((2, _CB), jnp.int32),
        pltpu.VMEM((2, _CB), jnp.int32),
        pltpu.VMEM((2, _CB), jnp.int32),
        pltpu.VMEM((2, _CB, _D), jnp.float32),
        pltpu.VMEM((2, _CB, _D), jnp.float32),
        pltpu.VMEM((2, _CB, _D), jnp.float32),
        pltpu.VMEM((_CB, 4 * _D), jnp.float32),
        pltpu.SemaphoreType.DMA,
        pltpu.SemaphoreType.DMA,
    ],
    compiler_params=pltpu.CompilerParams(use_tc_tiling_on_sc=False),
)
def _sc_embed(uid_hbm, age_hbm, gen_hbm, clicks_hbm, ut_hbm, at_hbm, gt_hbm,
              it_hbm, out_hbm, idx_v, rows_v, uidx_v, aidx_v, gidx_v,
              urows_v, arows_v, grows_v, outc_v, sem0, sem1):
    wid = lax.axis_index("s") * _NC + lax.axis_index("c")
    base = wid * _BPW
    inv = jnp.full((16,), 1.0 / _L, dtype=jnp.float32)
    sems = (sem0, sem1)

    def stage_and_fire(n, p, sem):
        rowbase = pl.multiple_of(base + n * _CB, _CB)
        ibase = pl.multiple_of(rowbase * _L, _CL)
        pltpu.sync_copy(clicks_hbm.at[pl.ds(ibase, _CL)], idx_v.at[p])
        for off, ln in _SLICES:
            pltpu.async_copy(
                it_hbm.at[idx_v.at[p].at[pl.ds(off, ln)]],
                rows_v.at[p].at[pl.ds(off, ln)], sem)
        pltpu.sync_copy(uid_hbm.at[pl.ds(rowbase, _CB)], uidx_v.at[p])
        pltpu.sync_copy(age_hbm.at[pl.ds(rowbase, _CB)], aidx_v.at[p])
        pltpu.sync_copy(gen_hbm.at[pl.ds(rowbase, _CB)], gidx_v.at[p])
        pltpu.async_copy(ut_hbm.at[uidx_v.at[p]], urows_v.at[p], sem)
        pltpu.async_copy(at_hbm.at[aidx_v.at[p]], arows_v.at[p], sem)
        pltpu.async_copy(gt_hbm.at[gidx_v.at[p]], grows_v.at[p], sem)

    def drain(p, sem):
        for off, ln in _SLICES:
            pltpu.make_async_copy(
                it_hbm.at[pl.ds(0, ln)], rows_v.at[p].at[pl.ds(off, ln)],
                sem).wait()
        pltpu.make_async_copy(ut_hbm.at[pl.ds(0, _CB)], urows_v.at[p],
                              sem).wait()
        pltpu.make_async_copy(ut_hbm.at[pl.ds(0, _CB)], arows_v.at[p],
                              sem).wait()
        pltpu.make_async_copy(ut_hbm.at[pl.ds(0, _CB)], grows_v.at[p],
                              sem).wait()

    def pool_and_store(n, p):
        rowbase = pl.multiple_of(base + n * _CB, _CB)

        def row_body(r, rcarry):
            slab = rows_v.at[p].at[pl.ds(r * _L, _L)]
            urow = urows_v.at[p].at[r]
            arow = arows_v.at[p].at[r]
            grow = grows_v.at[p].at[r]
            orow = outc_v.at[r]
            a0 = jnp.zeros((16,), jnp.float32)
            b0 = jnp.zeros((16,), jnp.float32)
            a1 = jnp.zeros((16,), jnp.float32)
            b1 = jnp.zeros((16,), jnp.float32)
            for jj in range(_L):
                lo = slab[jj, pl.ds(0, 16)]
                hi = slab[jj, pl.ds(16, 16)]
                if jj % 2 == 0:
                    a0 = a0 + lo
                    a1 = a1 + hi
                else:
                    b0 = b0 + lo
                    b1 = b1 + hi
            orow[pl.ds(0, 16)] = urow[pl.ds(0, 16)]
            orow[pl.ds(16, 16)] = urow[pl.ds(16, 16)]
            orow[pl.ds(32, 16)] = arow[pl.ds(0, 16)]
            orow[pl.ds(48, 16)] = arow[pl.ds(16, 16)]
            orow[pl.ds(64, 16)] = grow[pl.ds(0, 16)]
            orow[pl.ds(80, 16)] = grow[pl.ds(16, 16)]
            orow[pl.ds(96, 16)] = (a0 + b0) * inv
            orow[pl.ds(112, 16)] = (a1 + b1) * inv
            return rcarry

        lax.fori_loop(0, _CB, row_body, 0)
        pltpu.sync_copy(outc_v, out_hbm.at[pl.ds(rowbase, _CB)])

    stage_and_fire(0, 0, sems[0])

    def pair_body(t, carry):
        n0 = t * 2
        stage_and_fire(n0 + 1, 1, sems[1])
        drain(0, sems[0])
        pool_and_store(n0, 0)

        @pl.when(t < _NCHUNK // 2 - 1)
        def _():
            stage_and_fire(n0 + 2, 0, sems[0])

        drain(1, sems[1])
        pool_and_store(n0 + 1, 1)
        return carry

    lax.fori_loop(0, _NCHUNK // 2, pair_body, 0)


def _dense_body(x_ref, w_ref, b_ref, o_ref):
    acc = jnp.dot(x_ref[...], w_ref[...], preferred_element_type=jnp.float32)
    o_ref[...] = jnp.maximum(acc + b_ref[...], 0.0)


_BM = 1024
_dense = pl.pallas_call(
    _dense_body,
    grid=(_B // _BM,),
    in_specs=[
        pl.BlockSpec((_BM, 4 * _D), lambda i: (i, 0)),
        pl.BlockSpec((4 * _D, 64), lambda i: (0, 0)),
        pl.BlockSpec((1, 64), lambda i: (0, 0)),
    ],
    out_specs=pl.BlockSpec((_BM, 64), lambda i: (i, 0)),
    out_shape=jax.ShapeDtypeStruct((_B, 64), jnp.float32),
)


def kernel(user_id, age, gender, recent_clicks, user_table, age_table,
           gender_table, item_table, W, b):
    clicks_flat = recent_clicks.reshape(_B * _L)
    concat = _sc_embed(user_id, age, gender, clicks_flat, user_table,
                       age_table, gender_table, item_table)
    return _dense(concat, W, b.reshape(1, 64))
